# Initial kernel scaffold; baseline (speedup 1.0000x reference)
import jax
import jax.numpy as jnp
from jax.experimental import pallas as pl


def kernel(pillar_features, coords_batch, coords_spatial):
    raise NotImplementedError("write your pallas kernel here")


# speedup vs baseline: 2.6284x; 2.6284x over previous
import functools

import jax
import jax.numpy as jnp
from jax import lax
from jax.experimental import pallas as pl
from jax.experimental.pallas import tpu as pltpu
from jax.experimental.pallas import tpu_sc as plsc

C = 64
B = 4
NCB = 512 * 512
NCELL = B * NCB
P = 40000
TBL = 40960
PAD = P

CH = 4096
NCHUNK = NCELL // CH
CPB = NCB // CH
VECS = CH // 16

T0 = 0
T1 = TBL
IDXB = 2 * TBL
OUTB = IDXB + 2 * CH
POOL = OUTB + 4 * CH

_mesh = plsc.VectorSubcoreMesh(
    core_axis_name="c", subcore_axis_name="s", num_cores=2, num_subcores=16
)


@functools.partial(
    pl.kernel,
    out_type=jax.ShapeDtypeStruct((B * C * NCB,), jnp.int32),
    mesh=_mesh,
    compiler_params=pltpu.CompilerParams(needs_layout_passes=False),
    scratch_types=[
        pltpu.VMEM((POOL,), jnp.int32),
        pltpu.SemaphoreType.DMA,
        pltpu.SemaphoreType.DMA,
        pltpu.SemaphoreType.DMA,
        pltpu.SemaphoreType.DMA,
    ],
)
def _scatter_as_gather(tbl_hbm, idx_hbm, out_hbm, pool, si0, si1, so0, so1):
    wid = lax.axis_index("s") * 2 + lax.axis_index("c")
    c0 = wid * 2
    c1 = c0 + 1

    pltpu.sync_copy(tbl_hbm.at[pl.ds(c0 * TBL, TBL)], pool.at[pl.ds(T0, TBL)])
    pltpu.sync_copy(tbl_hbm.at[pl.ds(c1 * TBL, TBL)], pool.at[pl.ds(T1, TBL)])

    si = (si0, si1)
    so = (so0, so1)

    pltpu.async_copy(idx_hbm.at[pl.ds(0, CH)], pool.at[pl.ds(IDXB, CH)], si0)
    pltpu.async_copy(idx_hbm.at[pl.ds(CH, CH)], pool.at[pl.ds(IDXB + CH, CH)], si1)

    @pl.loop(0, NCHUNK, step=2)
    def _outer(q0):
        for r in range(2):
            q = q0 + r
            ib = IDXB + r * CH
            ob0 = OUTB + (2 * r) * CH
            ob1 = OUTB + (2 * r + 1) * CH

            bq = q // CPB
            s0 = (q % CPB) * CH
            off0 = (bq * C + c0) * NCB + s0
            off1 = (bq * C + c1) * NCB + s0

            pltpu.make_async_copy(
                idx_hbm.at[pl.ds(q * CH, CH)], pool.at[pl.ds(ib, CH)], si[r]
            ).wait()

            @pl.when(q >= 2)
            def _():
                pltpu.make_async_copy(
                    pool.at[pl.ds(ob0, CH)], out_hbm.at[pl.ds(off0, CH)], so[r]
                ).wait()
                pltpu.make_async_copy(
                    pool.at[pl.ds(ob1, CH)], out_hbm.at[pl.ds(off1, CH)], so[r]
                ).wait()

            @pl.loop(0, VECS, unroll=8)
            def _gather(i):
                iv = pool[pl.ds(ib + i * 16, 16)]
                g0 = plsc.load_gather(pool, [iv])
                g1 = plsc.load_gather(pool, [iv + TBL])
                pool[pl.ds(ob0 + i * 16, 16)] = g0
                pool[pl.ds(ob1 + i * 16, 16)] = g1

            pltpu.async_copy(
                pool.at[pl.ds(ob0, CH)], out_hbm.at[pl.ds(off0, CH)], so[r]
            )
            pltpu.async_copy(
                pool.at[pl.ds(ob1, CH)], out_hbm.at[pl.ds(off1, CH)], so[r]
            )

            @pl.when(q + 2 < NCHUNK)
            def _():
                pltpu.async_copy(
                    idx_hbm.at[pl.ds((q + 2) * CH, CH)],
                    pool.at[pl.ds(ib, CH)],
                    si[r],
                )

    for r in range(2):
        q = NCHUNK - 2 + r
        bq = q // CPB
        s0 = (q % CPB) * CH
        for j, ob in ((0, OUTB + 2 * r * CH), (1, OUTB + (2 * r + 1) * CH)):
            off = (bq * C + (c0 if j == 0 else c1)) * NCB + s0
            pltpu.make_async_copy(
                pool.at[pl.ds(ob, CH)], out_hbm.at[pl.ds(off, CH)], so[r]
            ).wait()


def kernel(pillar_features, coords_batch, coords_spatial):
    feat = pillar_features.reshape(C, P)
    featpad = jnp.pad(feat, ((0, 0), (0, TBL - P)))
    tbl_bits = lax.bitcast_convert_type(featpad, jnp.int32).reshape(-1)

    cell = coords_batch * NCB + coords_spatial
    tgt = jnp.where(coords_spatial > 0, cell, NCELL)
    pid = jnp.arange(P, dtype=jnp.int32)
    idx = jnp.full((NCELL + 1,), -1, jnp.int32).at[tgt].max(pid)
    idx = jnp.where(idx < 0, PAD, idx)[:NCELL]

    out_bits = _scatter_as_gather(tbl_bits, idx)
    out = lax.bitcast_convert_type(out_bits.reshape(B, C, NCB), jnp.float32)
    return out.reshape(B, C, 512, 512)


# speedup vs baseline: 2.7224x; 1.0358x over previous
import functools

import jax
import jax.numpy as jnp
from jax import lax
from jax.experimental import pallas as pl
from jax.experimental.pallas import tpu as pltpu
from jax.experimental.pallas import tpu_sc as plsc

C = 64
B = 4
NCB = 512 * 512
NCELL = B * NCB
P = 40000
TBL = 40960
PAD = P

CH = 4096
NCHUNK = NCELL // CH
CPB = NCB // CH
VECS = CH // 16

T0 = 0
T1 = TBL
IDXB = 2 * TBL
OUTB = IDXB + 2 * CH
POOL = OUTB + 4 * CH

NW = 32
OWN = NCELL // NW
NP = TBL
PCH = 2048
NPCH = NP // PCH
CB = OWN
CS = CB + 2 * PCH
POOLA = CS + 2 * PCH
SENT = -1

_mesh = plsc.VectorSubcoreMesh(
    core_axis_name="c", subcore_axis_name="s", num_cores=2, num_subcores=16
)


@functools.partial(
    pl.kernel,
    out_type=jax.ShapeDtypeStruct((B * C * NCB,), jnp.int32),
    mesh=_mesh,
    compiler_params=pltpu.CompilerParams(needs_layout_passes=False),
    scratch_types=[
        pltpu.VMEM((POOL,), jnp.int32),
        pltpu.SemaphoreType.DMA,
        pltpu.SemaphoreType.DMA,
        pltpu.SemaphoreType.DMA,
        pltpu.SemaphoreType.DMA,
    ],
)
def _scatter_as_gather(tbl_hbm, idx_hbm, out_hbm, pool, si0, si1, so0, so1):
    wid = lax.axis_index("s") * 2 + lax.axis_index("c")
    c0 = wid * 2
    c1 = c0 + 1

    pltpu.sync_copy(tbl_hbm.at[pl.ds(c0 * TBL, TBL)], pool.at[pl.ds(T0, TBL)])
    pltpu.sync_copy(tbl_hbm.at[pl.ds(c1 * TBL, TBL)], pool.at[pl.ds(T1, TBL)])

    si = (si0, si1)
    so = (so0, so1)

    pltpu.async_copy(idx_hbm.at[pl.ds(0, CH)], pool.at[pl.ds(IDXB, CH)], si0)
    pltpu.async_copy(idx_hbm.at[pl.ds(CH, CH)], pool.at[pl.ds(IDXB + CH, CH)], si1)

    @pl.loop(0, NCHUNK, step=2)
    def _outer(q0):
        for r in range(2):
            q = q0 + r
            ib = IDXB + r * CH
            ob0 = OUTB + (2 * r) * CH
            ob1 = OUTB + (2 * r + 1) * CH

            bq = q // CPB
            s0 = (q % CPB) * CH
            off0 = (bq * C + c0) * NCB + s0
            off1 = (bq * C + c1) * NCB + s0

            pltpu.make_async_copy(
                idx_hbm.at[pl.ds(q * CH, CH)], pool.at[pl.ds(ib, CH)], si[r]
            ).wait()

            @pl.when(q >= 2)
            def _():
                pltpu.make_async_copy(
                    pool.at[pl.ds(ob0, CH)], out_hbm.at[pl.ds(off0, CH)], so[r]
                ).wait()
                pltpu.make_async_copy(
                    pool.at[pl.ds(ob1, CH)], out_hbm.at[pl.ds(off1, CH)], so[r]
                ).wait()

            @pl.loop(0, VECS, unroll=8)
            def _gather(i):
                iv = pool[pl.ds(ib + i * 16, 16)]
                g0 = plsc.load_gather(pool, [iv])
                g1 = plsc.load_gather(pool, [iv + TBL])
                pool[pl.ds(ob0 + i * 16, 16)] = g0
                pool[pl.ds(ob1 + i * 16, 16)] = g1

            pltpu.async_copy(
                pool.at[pl.ds(ob0, CH)], out_hbm.at[pl.ds(off0, CH)], so[r]
            )
            pltpu.async_copy(
                pool.at[pl.ds(ob1, CH)], out_hbm.at[pl.ds(off1, CH)], so[r]
            )

            @pl.when(q + 2 < NCHUNK)
            def _():
                pltpu.async_copy(
                    idx_hbm.at[pl.ds((q + 2) * CH, CH)],
                    pool.at[pl.ds(ib, CH)],
                    si[r],
                )

    for r in range(2):
        q = NCHUNK - 2 + r
        bq = q // CPB
        s0 = (q % CPB) * CH
        for j, ob in ((0, OUTB + 2 * r * CH), (1, OUTB + (2 * r + 1) * CH)):
            off = (bq * C + (c0 if j == 0 else c1)) * NCB + s0
            pltpu.make_async_copy(
                pool.at[pl.ds(ob, CH)], out_hbm.at[pl.ds(off, CH)], so[r]
            ).wait()


@functools.partial(
    pl.kernel,
    out_type=jax.ShapeDtypeStruct((NCELL,), jnp.int32),
    mesh=_mesh,
    compiler_params=pltpu.CompilerParams(needs_layout_passes=False),
    scratch_types=[
        pltpu.VMEM((POOLA,), jnp.int32),
        pltpu.SemaphoreType.DMA,
        pltpu.SemaphoreType.DMA,
    ],
)
def _build_idx(cb_hbm, cs_hbm, idx_hbm, pool, sa0, sa1):
    wid = lax.axis_index("s") * 2 + lax.axis_index("c")
    base = wid * OWN
    sa = (sa0, sa1)
    iota = lax.iota(jnp.int32, 16)
    padv = jnp.full((16,), PAD, jnp.int32)

    @pl.loop(0, OWN // 16, unroll=8)
    def _clear(i):
        pool[pl.ds(i * 16, 16)] = padv

    for r in range(2):
        pltpu.async_copy(
            cb_hbm.at[pl.ds(r * PCH, PCH)], pool.at[pl.ds(CB + r * PCH, PCH)], sa[r]
        )
        pltpu.async_copy(
            cs_hbm.at[pl.ds(r * PCH, PCH)], pool.at[pl.ds(CS + r * PCH, PCH)], sa[r]
        )

    @pl.loop(0, NPCH, step=2)
    def _outer(k0):
        for r in range(2):
            k = k0 + r
            cb = CB + r * PCH
            cs = CS + r * PCH
            pltpu.make_async_copy(
                cb_hbm.at[pl.ds(k * PCH, PCH)], pool.at[pl.ds(cb, PCH)], sa[r]
            ).wait()
            pltpu.make_async_copy(
                cs_hbm.at[pl.ds(k * PCH, PCH)], pool.at[pl.ds(cs, PCH)], sa[r]
            ).wait()

            @pl.loop(0, PCH // 16)
            def _pts(i):
                bb = pool[pl.ds(cb + i * 16, 16)]
                ss = pool[pl.ds(cs + i * 16, 16)]
                local = bb * NCB + ss - base
                pid = k * PCH + i * 16 + iota
                valid = (ss > 0) & (local >= 0) & (local < OWN)
                key = jnp.where(valid, (local << 16) | pid, SENT)
                sk, sv = plsc.sort_key_val(key, pid)
                nxt = sk.at[jnp.minimum(iota + 1, 15)].get(
                    mode="promise_in_bounds")
                win = ((sk >> 16) != (nxt >> 16)) | (iota == 15)
                plsc.store_scatter(
                    pool, [sk >> 16], sv, mask=win & (sk >= 0)
                )

            @pl.when(k + 2 < NPCH)
            def _():
                pltpu.async_copy(
                    cb_hbm.at[pl.ds((k + 2) * PCH, PCH)],
                    pool.at[pl.ds(cb, PCH)],
                    sa[r],
                )
                pltpu.async_copy(
                    cs_hbm.at[pl.ds((k + 2) * PCH, PCH)],
                    pool.at[pl.ds(cs, PCH)],
                    sa[r],
                )

    pltpu.sync_copy(pool.at[pl.ds(0, OWN)], idx_hbm.at[pl.ds(base, OWN)])


def kernel(pillar_features, coords_batch, coords_spatial):
    feat = pillar_features.reshape(C, P)
    featpad = jnp.pad(feat, ((0, 0), (0, TBL - P)))
    tbl_bits = lax.bitcast_convert_type(featpad, jnp.int32).reshape(-1)

    cbp = jnp.pad(coords_batch, (0, NP - P))
    csp = jnp.pad(coords_spatial, (0, NP - P))
    idx = _build_idx(cbp, csp)

    out_bits = _scatter_as_gather(tbl_bits, idx)
    out = lax.bitcast_convert_type(out_bits.reshape(B, C, NCB), jnp.float32)
    return out.reshape(B, C, 512, 512)


# speedup vs baseline: 3.4213x; 1.2567x over previous
import functools

import jax
import jax.numpy as jnp
from jax import lax
from jax.experimental import pallas as pl
from jax.experimental.pallas import tpu as pltpu
from jax.experimental.pallas import tpu_sc as plsc

C = 64
B = 4
NCB = 512 * 512
NCELL = B * NCB
P = 40000
TW = P

CH = 4096
NCHUNK = NCELL // CH
CPB = NCB // CH
LCAP = 512

NW = 32
OWN = NCELL // NW
CHPT = OWN // CH
PCH = 2000
NPCH = P // PCH

SENT = -1

A_BLK = 0
A_CB = A_BLK + OWN
A_CS = A_CB + 2 * PCH
A_LST = A_CS + 2 * PCH
A_CNT = A_LST + 2 * CH
POOLA = A_CNT + CHPT * 16

B_T0 = 0
B_T1 = TW
B_CNT = 2 * TW
B_LST = B_CNT + NCHUNK * 16
B_SPARE = B_LST + 4 * LCAP
B_OUT = B_SPARE + LCAP
POOLB = B_OUT + 4 * CH

_mesh = plsc.VectorSubcoreMesh(
    core_axis_name="c", subcore_axis_name="s", num_cores=2, num_subcores=16
)
_params = pltpu.CompilerParams(needs_layout_passes=False)


@functools.partial(
    pl.kernel,
    out_type=(
        jax.ShapeDtypeStruct((NCHUNK * CH,), jnp.int32),
        jax.ShapeDtypeStruct((NCHUNK * 16,), jnp.int32),
    ),
    mesh=_mesh,
    compiler_params=_params,
    scratch_types=[
        pltpu.VMEM((POOLA,), jnp.int32),
        pltpu.SemaphoreType.DMA,
        pltpu.SemaphoreType.DMA,
        pltpu.SemaphoreType.DMA,
        pltpu.SemaphoreType.DMA,
    ],
)
def _build_lists(cb_hbm, cs_hbm, lists_hbm, counts_hbm, pool, sa0, sa1, sl0, sl1):
    wid = lax.axis_index("s") * 2 + lax.axis_index("c")
    base = wid * OWN
    sa = (sa0, sa1)
    sl = (sl0, sl1)
    iota = lax.iota(jnp.int32, 16)
    emptyv = jnp.full((16,), -1, jnp.int32)

    @pl.loop(0, OWN // 16, unroll=8)
    def _clear(i):
        pool[pl.ds(A_BLK + i * 16, 16)] = emptyv

    for r in range(2):
        pltpu.async_copy(
            cb_hbm.at[pl.ds(r * PCH, PCH)], pool.at[pl.ds(A_CB + r * PCH, PCH)], sa[r]
        )
        pltpu.async_copy(
            cs_hbm.at[pl.ds(r * PCH, PCH)], pool.at[pl.ds(A_CS + r * PCH, PCH)], sa[r]
        )

    @pl.loop(0, NPCH, step=2)
    def _outer(k0):
        for r in range(2):
            k = k0 + r
            cb = A_CB + r * PCH
            cs = A_CS + r * PCH
            pltpu.make_async_copy(
                cb_hbm.at[pl.ds(k * PCH, PCH)], pool.at[pl.ds(cb, PCH)], sa[r]
            ).wait()
            pltpu.make_async_copy(
                cs_hbm.at[pl.ds(k * PCH, PCH)], pool.at[pl.ds(cs, PCH)], sa[r]
            ).wait()

            @pl.loop(0, PCH // 16)
            def _pts(i):
                bb = pool[pl.ds(cb + i * 16, 16)]
                ss = pool[pl.ds(cs + i * 16, 16)]
                local = bb * NCB + ss - base
                pid = k * PCH + i * 16 + iota
                valid = (ss > 0) & (local >= 0) & (local < OWN)
                key = jnp.where(valid, (local << 16) | pid, SENT)
                sk, sv = plsc.sort_key_val(key, pid)
                nxt = sk.at[jnp.minimum(iota + 1, 15)].get(
                    mode="promise_in_bounds")
                win = ((sk >> 16) != (nxt >> 16)) | (iota == 15)
                plsc.store_scatter(
                    pool, [A_BLK + (sk >> 16)], sv, mask=win & (sk >= 0)
                )

            @pl.when(k + 2 < NPCH)
            def _():
                pltpu.async_copy(
                    cb_hbm.at[pl.ds((k + 2) * PCH, PCH)],
                    pool.at[pl.ds(cb, PCH)],
                    sa[r],
                )
                pltpu.async_copy(
                    cs_hbm.at[pl.ds((k + 2) * PCH, PCH)],
                    pool.at[pl.ds(cs, PCH)],
                    sa[r],
                )

    for j in range(CHPT):
        r = j % 2
        if j >= 2:
            pltpu.make_async_copy(
                pool.at[pl.ds(A_LST + r * CH, CH)],
                lists_hbm.at[pl.ds((wid * CHPT + j - 2) * CH, CH)],
                sl[r],
            ).wait()

        @pl.loop(0, CH // 16, init_carry=jnp.int32(0))
        def _compress(i, wpos):
            v = pool[pl.ds(A_BLK + j * CH + i * 16, 16)]
            m = v >= 0
            val = ((i * 16 + iota) << 16) | v
            plsc.store_compressed(
                pool.at[pl.ds(A_LST + r * CH + wpos, 16)], val, mask=m
            )
            return wpos + jnp.max(plsc.all_reduce_population_count(m))

        cnt = _compress
        pool[pl.ds(A_CNT + j * 16, 16)] = jnp.broadcast_to(cnt, (16,))
        pltpu.async_copy(
            pool.at[pl.ds(A_LST + r * CH, CH)],
            lists_hbm.at[pl.ds((wid * CHPT + j) * CH, CH)],
            sl[r],
        )

    for j in (CHPT - 2, CHPT - 1):
        pltpu.make_async_copy(
            pool.at[pl.ds(A_LST + (j % 2) * CH, CH)],
            lists_hbm.at[pl.ds((wid * CHPT + j) * CH, CH)],
            sl[j % 2],
        ).wait()

    pltpu.sync_copy(
        pool.at[pl.ds(A_CNT, CHPT * 16)],
        counts_hbm.at[pl.ds(wid * CHPT * 16, CHPT * 16)],
    )


@functools.partial(
    pl.kernel,
    out_type=jax.ShapeDtypeStruct((B * C * NCB,), jnp.int32),
    mesh=_mesh,
    compiler_params=_params,
    scratch_types=[
        pltpu.VMEM((POOLB,), jnp.int32),
        pltpu.SemaphoreType.DMA,
        pltpu.SemaphoreType.DMA,
        pltpu.SemaphoreType.DMA,
        pltpu.SemaphoreType.DMA,
        pltpu.SemaphoreType.DMA,
        pltpu.SemaphoreType.DMA,
    ],
)
def _emit(tbl_hbm, lists_hbm, counts_hbm, out_hbm, pool,
          sl0, sl1, sl2, sl3, so0, so1):
    wid = lax.axis_index("s") * 2 + lax.axis_index("c")
    c0 = wid * 2
    c1 = c0 + 1
    sl = (sl0, sl1, sl2, sl3)
    so = (so0, so1)
    iota = lax.iota(jnp.int32, 16)
    zerov = jnp.zeros((16,), jnp.int32)

    pltpu.sync_copy(tbl_hbm.at[pl.ds(c0 * TW, TW)], pool.at[pl.ds(B_T0, TW)])
    pltpu.sync_copy(tbl_hbm.at[pl.ds(c1 * TW, TW)], pool.at[pl.ds(B_T1, TW)])
    pltpu.sync_copy(counts_hbm, pool.at[pl.ds(B_CNT, NCHUNK * 16)])

    @pl.loop(0, 4 * CH // 16, unroll=8)
    def _zero(i):
        pool[pl.ds(B_OUT + i * 16, 16)] = zerov

    for q in range(2):
        pltpu.async_copy(
            lists_hbm.at[pl.ds(q * CH, LCAP)],
            pool.at[pl.ds(B_LST + q * LCAP, LCAP)],
            sl[q],
        )

    @pl.loop(0, NCHUNK, step=4)
    def _chunks(q4):
        for r4 in range(4):
            q = q4 + r4
            pr = r4 % 2
            ob0 = B_OUT + (2 * pr) * CH
            ob1 = B_OUT + (2 * pr + 1) * CH
            lst = B_LST + r4 * LCAP

            bq = q // CPB
            s0 = (q % CPB) * CH
            off0 = (bq * C + c0) * NCB + s0
            off1 = (bq * C + c1) * NCB + s0

            cnt = jnp.max(pool[pl.ds(B_CNT + q * 16, 16)])

            @pl.when(q >= 2)
            def _():
                pltpu.make_async_copy(
                    pool.at[pl.ds(ob0, CH)], out_hbm.at[pl.ds(off0, CH)], so[pr]
                ).wait()
                pltpu.make_async_copy(
                    pool.at[pl.ds(ob1, CH)], out_hbm.at[pl.ds(off1, CH)], so[pr]
                ).wait()
                pcnt = jnp.max(pool[pl.ds(B_CNT + (q - 2) * 16, 16)])
                pfast = jnp.minimum(pcnt, LCAP)
                plst = B_LST + ((r4 + 2) % 4) * LCAP

                @pl.loop(0, (pfast + 15) // 16)
                def _rz(i):
                    wv = pool[pl.ds(plst + i * 16, 16)]
                    m = (i * 16 + iota) < pfast
                    loc = jnp.where(m, wv, 0) >> 16
                    plsc.store_scatter(pool, [ob0 + loc], zerov, mask=m)
                    plsc.store_scatter(pool, [ob1 + loc], zerov, mask=m)

                @pl.when(pcnt > LCAP)
                def _():
                    @pl.loop(0, (pcnt - LCAP + LCAP - 1) // LCAP)
                    def _seg(g):
                        pltpu.sync_copy(
                            lists_hbm.at[
                                pl.ds((q - 2) * CH + LCAP + g * LCAP, LCAP)
                            ],
                            pool.at[pl.ds(B_SPARE, LCAP)],
                        )
                        rem = pcnt - LCAP - g * LCAP

                        @pl.loop(0, LCAP // 16)
                        def _rzs(i):
                            wv = pool[pl.ds(B_SPARE + i * 16, 16)]
                            m = (i * 16 + iota) < rem
                            loc = jnp.where(m, wv, 0) >> 16
                            plsc.store_scatter(pool, [ob0 + loc], zerov, mask=m)
                            plsc.store_scatter(pool, [ob1 + loc], zerov, mask=m)

            pltpu.make_async_copy(
                lists_hbm.at[pl.ds(q * CH, LCAP)],
                pool.at[pl.ds(lst, LCAP)],
                sl[r4],
            ).wait()
            ncnt = jnp.minimum(cnt, LCAP)

            @pl.loop(0, (ncnt + 15) // 16)
            def _sc(i):
                wv = pool[pl.ds(lst + i * 16, 16)]
                m = (i * 16 + iota) < ncnt
                wv = jnp.where(m, wv, 0)
                loc = wv >> 16
                pid = wv & 0xFFFF
                g0 = plsc.load_gather(pool, [B_T0 + pid])
                g1 = plsc.load_gather(pool, [B_T1 + pid])
                plsc.store_scatter(pool, [ob0 + loc], g0, mask=m)
                plsc.store_scatter(pool, [ob1 + loc], g1, mask=m)

            @pl.when(cnt > LCAP)
            def _():
                @pl.loop(0, (cnt - LCAP + LCAP - 1) // LCAP)
                def _seg(g):
                    pltpu.sync_copy(
                        lists_hbm.at[pl.ds(q * CH + LCAP + g * LCAP, LCAP)],
                        pool.at[pl.ds(B_SPARE, LCAP)],
                    )
                    rem = cnt - LCAP - g * LCAP

                    @pl.loop(0, LCAP // 16)
                    def _scs(i):
                        wv = pool[pl.ds(B_SPARE + i * 16, 16)]
                        m = (i * 16 + iota) < rem
                        wv = jnp.where(m, wv, 0)
                        loc = wv >> 16
                        pid = wv & 0xFFFF
                        g0 = plsc.load_gather(pool, [B_T0 + pid])
                        g1 = plsc.load_gather(pool, [B_T1 + pid])
                        plsc.store_scatter(pool, [ob0 + loc], g0, mask=m)
                        plsc.store_scatter(pool, [ob1 + loc], g1, mask=m)

            pltpu.async_copy(
                pool.at[pl.ds(ob0, CH)], out_hbm.at[pl.ds(off0, CH)], so[pr]
            )
            pltpu.async_copy(
                pool.at[pl.ds(ob1, CH)], out_hbm.at[pl.ds(off1, CH)], so[pr]
            )

            @pl.when(q + 2 < NCHUNK)
            def _():
                pltpu.async_copy(
                    lists_hbm.at[pl.ds((q + 2) * CH, LCAP)],
                    pool.at[pl.ds(B_LST + ((r4 + 2) % 4) * LCAP, LCAP)],
                    sl[(r4 + 2) % 4],
                )

    for r4 in (2, 3):
        q = NCHUNK - 2 + (r4 - 2)
        pr = r4 % 2
        bq = q // CPB
        s0 = (q % CPB) * CH
        for cc, ob in ((c0, B_OUT + 2 * pr * CH), (c1, B_OUT + (2 * pr + 1) * CH)):
            off = (bq * C + cc) * NCB + s0
            pltpu.make_async_copy(
                pool.at[pl.ds(ob, CH)], out_hbm.at[pl.ds(off, CH)], so[pr]
            ).wait()


def kernel(pillar_features, coords_batch, coords_spatial):
    tbl_bits = lax.bitcast_convert_type(
        pillar_features.reshape(C, P), jnp.int32
    ).reshape(-1)

    lists, counts = _build_lists(coords_batch, coords_spatial)
    out_bits = _emit(tbl_bits, lists, counts)
    out = lax.bitcast_convert_type(out_bits.reshape(B, C, NCB), jnp.float32)
    return out.reshape(B, C, 512, 512)


# speedup vs baseline: 8.8327x; 2.5817x over previous
import functools

import jax
import jax.numpy as jnp
from jax import lax
from jax.experimental import pallas as pl
from jax.experimental.pallas import tpu as pltpu
from jax.experimental.pallas import tpu_sc as plsc

C = 64
B = 4
NCB = 512 * 512
NCELL = B * NCB
P = 40000
TW = P

CH = 4096
NCHUNK = NCELL // CH
CPB = NCB // CH
LCAP = 512

NW = 32
OWN = NCELL // NW
CHPT = OWN // CH
PCH = 2000
NPCH = P // PCH

SENT = -1

A_BLK = 0
A_CB = A_BLK + OWN
A_CS = A_CB + 2 * PCH
A_LST = A_CS + 2 * PCH
A_CNT = A_LST + 2 * CH
POOLA = A_CNT + CHPT * 16

B_T0 = 0
B_T1 = TW
POOLBF = 2 * TW
B_CNT = 0
B_LST = B_CNT + NCHUNK * 16
B_SPARE = B_LST + 4 * LCAP
POOLBI = B_SPARE + LCAP
ROWS = CH // 512

_mesh = plsc.VectorSubcoreMesh(
    core_axis_name="c", subcore_axis_name="s", num_cores=2, num_subcores=16
)
_params = pltpu.CompilerParams(needs_layout_passes=False)


@functools.partial(
    pl.kernel,
    out_type=(
        jax.ShapeDtypeStruct((NCHUNK * CH,), jnp.int32),
        jax.ShapeDtypeStruct((NCHUNK * 16,), jnp.int32),
    ),
    mesh=_mesh,
    compiler_params=_params,
    scratch_types=[
        pltpu.VMEM((POOLA,), jnp.int32),
        pltpu.SemaphoreType.DMA,
        pltpu.SemaphoreType.DMA,
        pltpu.SemaphoreType.DMA,
        pltpu.SemaphoreType.DMA,
    ],
)
def _build_lists(cb_hbm, cs_hbm, lists_hbm, counts_hbm, pool, sa0, sa1, sl0, sl1):
    wid = lax.axis_index("s") * 2 + lax.axis_index("c")
    base = wid * OWN
    sa = (sa0, sa1)
    sl = (sl0, sl1)
    iota = lax.iota(jnp.int32, 16)
    emptyv = jnp.full((16,), -1, jnp.int32)

    @pl.loop(0, OWN // 16, unroll=8)
    def _clear(i):
        pool[pl.ds(A_BLK + i * 16, 16)] = emptyv

    for r in range(2):
        pltpu.async_copy(
            cb_hbm.at[pl.ds(r * PCH, PCH)], pool.at[pl.ds(A_CB + r * PCH, PCH)], sa[r]
        )
        pltpu.async_copy(
            cs_hbm.at[pl.ds(r * PCH, PCH)], pool.at[pl.ds(A_CS + r * PCH, PCH)], sa[r]
        )

    @pl.loop(0, NPCH, step=2)
    def _outer(k0):
        for r in range(2):
            k = k0 + r
            cb = A_CB + r * PCH
            cs = A_CS + r * PCH
            pltpu.make_async_copy(
                cb_hbm.at[pl.ds(k * PCH, PCH)], pool.at[pl.ds(cb, PCH)], sa[r]
            ).wait()
            pltpu.make_async_copy(
                cs_hbm.at[pl.ds(k * PCH, PCH)], pool.at[pl.ds(cs, PCH)], sa[r]
            ).wait()

            @pl.loop(0, PCH // 16)
            def _pts(i):
                bb = pool[pl.ds(cb + i * 16, 16)]
                ss = pool[pl.ds(cs + i * 16, 16)]
                local = bb * NCB + ss - base
                pid = k * PCH + i * 16 + iota
                valid = (ss > 0) & (local >= 0) & (local < OWN)
                key = jnp.where(valid, (local << 16) | pid, SENT)
                sk, sv = plsc.sort_key_val(key, pid)
                nxt = sk.at[jnp.minimum(iota + 1, 15)].get(
                    mode="promise_in_bounds")
                win = ((sk >> 16) != (nxt >> 16)) | (iota == 15)
                plsc.store_scatter(
                    pool, [A_BLK + (sk >> 16)], sv, mask=win & (sk >= 0)
                )

            @pl.when(k + 2 < NPCH)
            def _():
                pltpu.async_copy(
                    cb_hbm.at[pl.ds((k + 2) * PCH, PCH)],
                    pool.at[pl.ds(cb, PCH)],
                    sa[r],
                )
                pltpu.async_copy(
                    cs_hbm.at[pl.ds((k + 2) * PCH, PCH)],
                    pool.at[pl.ds(cs, PCH)],
                    sa[r],
                )

    for j in range(CHPT):
        r = j % 2
        if j >= 2:
            pltpu.make_async_copy(
                pool.at[pl.ds(A_LST + r * CH, CH)],
                lists_hbm.at[pl.ds((wid * CHPT + j - 2) * CH, CH)],
                sl[r],
            ).wait()

        @pl.loop(0, CH // 16, init_carry=jnp.int32(0))
        def _compress(i, wpos):
            v = pool[pl.ds(A_BLK + j * CH + i * 16, 16)]
            m = v >= 0
            val = ((i * 16 + iota) << 16) | v
            plsc.store_compressed(
                pool.at[pl.ds(A_LST + r * CH + wpos, 16)], val, mask=m
            )
            return wpos + jnp.max(plsc.all_reduce_population_count(m))

        cnt = _compress
        pool[pl.ds(A_CNT + j * 16, 16)] = jnp.broadcast_to(cnt, (16,))
        pltpu.async_copy(
            pool.at[pl.ds(A_LST + r * CH, CH)],
            lists_hbm.at[pl.ds((wid * CHPT + j) * CH, CH)],
            sl[r],
        )

    for j in (CHPT - 2, CHPT - 1):
        pltpu.make_async_copy(
            pool.at[pl.ds(A_LST + (j % 2) * CH, CH)],
            lists_hbm.at[pl.ds((wid * CHPT + j) * CH, CH)],
            sl[j % 2],
        ).wait()

    pltpu.sync_copy(
        pool.at[pl.ds(A_CNT, CHPT * 16)],
        counts_hbm.at[pl.ds(wid * CHPT * 16, CHPT * 16)],
    )


@functools.partial(
    pl.kernel,
    out_type=jax.ShapeDtypeStruct((B, C, 512, 512), jnp.float32),
    mesh=_mesh,
    compiler_params=_params,
    scratch_types=[
        pltpu.VMEM((POOLBF,), jnp.float32),
        pltpu.VMEM((2, 2, ROWS, 512), jnp.float32),
        pltpu.VMEM((POOLBI,), jnp.int32),
        pltpu.SemaphoreType.DMA,
        pltpu.SemaphoreType.DMA,
        pltpu.SemaphoreType.DMA,
        pltpu.SemaphoreType.DMA,
        pltpu.SemaphoreType.DMA,
        pltpu.SemaphoreType.DMA,
    ],
)
def _emit(tbl_hbm, lists_hbm, counts_hbm, out_hbm, poolf, outf, pooli,
          sl0, sl1, sl2, sl3, so0, so1):
    wid = lax.axis_index("s") * 2 + lax.axis_index("c")
    c0 = wid * 2
    c1 = c0 + 1
    sl = (sl0, sl1, sl2, sl3)
    so = (so0, so1)
    iota = lax.iota(jnp.int32, 16)
    zerovf = jnp.zeros((16,), jnp.float32)

    pltpu.sync_copy(tbl_hbm.at[pl.ds(c0 * TW, TW)], poolf.at[pl.ds(B_T0, TW)])
    pltpu.sync_copy(tbl_hbm.at[pl.ds(c1 * TW, TW)], poolf.at[pl.ds(B_T1, TW)])
    pltpu.sync_copy(counts_hbm, pooli.at[pl.ds(B_CNT, NCHUNK * 16)])

    for pr in range(2):
        for j in range(2):

            @pl.loop(0, ROWS * 512 // 16, unroll=8)
            def _zero(i):
                outf[pr, j, i // 32, pl.ds((i % 32) * 16, 16)] = zerovf

    for q in range(2):
        pltpu.async_copy(
            lists_hbm.at[pl.ds(q * CH, LCAP)],
            pooli.at[pl.ds(B_LST + q * LCAP, LCAP)],
            sl[q],
        )

    @pl.loop(0, NCHUNK, step=4)
    def _chunks(q4):
        for r4 in range(4):
            q = q4 + r4
            pr = r4 % 2
            lst = B_LST + r4 * LCAP

            bq = q // CPB
            r0 = (q % CPB) * ROWS

            cnt = jnp.max(pooli[pl.ds(B_CNT + q * 16, 16)])

            @pl.when(q >= 2)
            def _():
                pltpu.make_async_copy(
                    outf.at[pr, 0],
                    out_hbm.at[bq, c0, pl.ds(r0, ROWS), :],
                    so[pr],
                ).wait()
                pltpu.make_async_copy(
                    outf.at[pr, 1],
                    out_hbm.at[bq, c1, pl.ds(r0, ROWS), :],
                    so[pr],
                ).wait()
                pcnt = jnp.max(pooli[pl.ds(B_CNT + (q - 2) * 16, 16)])
                pfast = jnp.minimum(pcnt, LCAP)
                plst = B_LST + ((r4 + 2) % 4) * LCAP

                @pl.loop(0, (pfast + 15) // 16)
                def _rz(i):
                    wv = pooli[pl.ds(plst + i * 16, 16)]
                    m = (i * 16 + iota) < pfast
                    loc = jnp.where(m, wv, 0) >> 16
                    row = loc >> 9
                    col = loc & 511
                    plsc.store_scatter(outf.at[pr, 0], [row, col], zerovf, mask=m)
                    plsc.store_scatter(outf.at[pr, 1], [row, col], zerovf, mask=m)

                @pl.when(pcnt > LCAP)
                def _():
                    @pl.loop(0, (pcnt - LCAP + LCAP - 1) // LCAP)
                    def _seg(g):
                        pltpu.sync_copy(
                            lists_hbm.at[
                                pl.ds((q - 2) * CH + LCAP + g * LCAP, LCAP)
                            ],
                            pooli.at[pl.ds(B_SPARE, LCAP)],
                        )
                        rem = pcnt - LCAP - g * LCAP

                        @pl.loop(0, LCAP // 16)
                        def _rzs(i):
                            wv = pooli[pl.ds(B_SPARE + i * 16, 16)]
                            m = (i * 16 + iota) < rem
                            loc = jnp.where(m, wv, 0) >> 16
                            row = loc >> 9
                            col = loc & 511
                            plsc.store_scatter(
                                outf.at[pr, 0], [row, col], zerovf, mask=m)
                            plsc.store_scatter(
                                outf.at[pr, 1], [row, col], zerovf, mask=m)

            pltpu.make_async_copy(
                lists_hbm.at[pl.ds(q * CH, LCAP)],
                pooli.at[pl.ds(lst, LCAP)],
                sl[r4],
            ).wait()
            ncnt = jnp.minimum(cnt, LCAP)

            @pl.loop(0, (ncnt + 15) // 16)
            def _sc(i):
                wv = pooli[pl.ds(lst + i * 16, 16)]
                m = (i * 16 + iota) < ncnt
                wv = jnp.where(m, wv, 0)
                loc = wv >> 16
                row = loc >> 9
                col = loc & 511
                pid = wv & 0xFFFF
                g0 = plsc.load_gather(poolf, [B_T0 + pid])
                g1 = plsc.load_gather(poolf, [B_T1 + pid])
                plsc.store_scatter(outf.at[pr, 0], [row, col], g0, mask=m)
                plsc.store_scatter(outf.at[pr, 1], [row, col], g1, mask=m)

            @pl.when(cnt > LCAP)
            def _():
                @pl.loop(0, (cnt - LCAP + LCAP - 1) // LCAP)
                def _seg(g):
                    pltpu.sync_copy(
                        lists_hbm.at[pl.ds(q * CH + LCAP + g * LCAP, LCAP)],
                        pooli.at[pl.ds(B_SPARE, LCAP)],
                    )
                    rem = cnt - LCAP - g * LCAP

                    @pl.loop(0, LCAP // 16)
                    def _scs(i):
                        wv = pooli[pl.ds(B_SPARE + i * 16, 16)]
                        m = (i * 16 + iota) < rem
                        wv = jnp.where(m, wv, 0)
                        loc = wv >> 16
                        row = loc >> 9
                        col = loc & 511
                        pid = wv & 0xFFFF
                        g0 = plsc.load_gather(poolf, [B_T0 + pid])
                        g1 = plsc.load_gather(poolf, [B_T1 + pid])
                        plsc.store_scatter(
                            outf.at[pr, 0], [row, col], g0, mask=m)
                        plsc.store_scatter(
                            outf.at[pr, 1], [row, col], g1, mask=m)

            pltpu.async_copy(
                outf.at[pr, 0], out_hbm.at[bq, c0, pl.ds(r0, ROWS), :], so[pr]
            )
            pltpu.async_copy(
                outf.at[pr, 1], out_hbm.at[bq, c1, pl.ds(r0, ROWS), :], so[pr]
            )

            @pl.when(q + 2 < NCHUNK)
            def _():
                pltpu.async_copy(
                    lists_hbm.at[pl.ds((q + 2) * CH, LCAP)],
                    pooli.at[pl.ds(B_LST + ((r4 + 2) % 4) * LCAP, LCAP)],
                    sl[(r4 + 2) % 4],
                )

    for r4 in (2, 3):
        q = NCHUNK - 2 + (r4 - 2)
        pr = r4 % 2
        bq = q // CPB
        r0 = (q % CPB) * ROWS
        for jj, cc in ((0, c0), (1, c1)):
            pltpu.make_async_copy(
                outf.at[pr, jj], out_hbm.at[bq, cc, pl.ds(r0, ROWS), :], so[pr]
            ).wait()


def kernel(pillar_features, coords_batch, coords_spatial):
    tbl = pillar_features.reshape(C * P)
    lists, counts = _build_lists(coords_batch, coords_spatial)
    return _emit(tbl, lists, counts)
